# baseline (speedup 1.0000x reference)
import functools

import jax
import jax.numpy as jnp
from jax import lax
from jax.experimental import pallas as pl
from jax.experimental.pallas import tpu as pltpu
from jax.experimental.pallas import tpu_sc as plsc

N_NODES = 10000
N_EDGES = 320000
DIM_IN = 128
H1 = 128
H2 = 64
DIM_OUT = 16

NC = 2
NS = 16
NW = NC * NS
C = 128
NCH0 = 59
NCH1 = 101
NCHM = max(NCH0, NCH1)
EPAD = NS * (NCH0 + NCH1) * C
N2 = 10112
STRIPE = N2 // NS

BN = 2528
GRID = 4

_MESH = dict(core_axis_name="c", subcore_axis_name="s")
DEGW = 128
NSTR = 4


def _make_agg(D):
    @functools.partial(
        pl.kernel,
        mesh=plsc.VectorSubcoreMesh(**_MESH),
        out_type=jax.ShapeDtypeStruct((NC, N2, D), jnp.float32),
        scratch_types=[
            pltpu.VMEM((NCHM, C), jnp.int32),
            pltpu.VMEM((NCHM, C), jnp.int32),
            pltpu.VMEM((C, D), jnp.float32),
            pltpu.VMEM_SHARED((N2, D), jnp.float32),
        ] + [pltpu.SemaphoreType.DMA] * NSTR,
    )
    def agg(f_hbm, src_hbm, dst_hbm, out_hbm, src_v, dst_v, buf0,
            table, *gsems):
        cid = lax.axis_index("c")
        sid = lax.axis_index("s")
        wid = sid * NC + cid
        pltpu.sync_copy(
            f_hbm.at[pl.ds(sid * STRIPE, STRIPE)],
            table.at[pl.ds(sid * STRIPE, STRIPE)],
        )
        plsc.subcore_barrier()

        pltpu.sync_copy(src_hbm.at[wid], src_v)
        pltpu.sync_copy(dst_hbm.at[wid], dst_v)

        CS = C // NSTR

        def body(k, carry):
            cps = [
                pltpu.async_copy(
                    f_hbm.at[src_v.at[k, pl.ds(j * CS, CS)]],
                    buf0.at[pl.ds(j * CS, CS)], gsems[j])
                for j in range(NSTR)
            ]
            for cp in cps:
                cp.wait()
            pltpu.sync_copy(buf0, table.at[dst_v.at[k]], add=True)
            return carry

        nch = jnp.where(cid == 0, NCH0, NCH1)
        lax.fori_loop(0, nch, body, 0)
        plsc.subcore_barrier()
        pltpu.sync_copy(
            table.at[pl.ds(sid * STRIPE, STRIPE)],
            out_hbm.at[cid, pl.ds(sid * STRIPE, STRIPE)],
        )

    return agg


@functools.partial(
    pl.kernel,
    mesh=plsc.VectorSubcoreMesh(**_MESH),
    out_type=jax.ShapeDtypeStruct((NC, N2, DEGW), jnp.float32),
    scratch_types=[
        pltpu.VMEM((NCHM, C), jnp.int32),
        pltpu.VMEM((C, DEGW), jnp.float32),
        pltpu.VMEM_SHARED((N2, DEGW), jnp.float32),
    ],
)
def _deg(dst_hbm, ones_hbm, out_hbm, dst_v, ones_v, table):
    cid = lax.axis_index("c")
    sid = lax.axis_index("s")
    wid = sid * NC + cid
    pltpu.sync_copy(dst_hbm.at[wid], dst_v)
    pltpu.sync_copy(ones_hbm.at[pl.ds(0, C)], ones_v)
    pltpu.sync_copy(
        ones_hbm.at[pl.ds(sid * STRIPE, STRIPE)],
        table.at[pl.ds(sid * STRIPE, STRIPE)],
    )
    plsc.subcore_barrier()

    def body(k, carry):
        pltpu.sync_copy(ones_v, table.at[dst_v.at[k]], add=True)
        return carry

    lax.fori_loop(0, jnp.where(cid == 0, NCH0, NCH1), body, 0)
    plsc.subcore_barrier()
    pltpu.sync_copy(
        table.at[pl.ds(sid * STRIPE, STRIPE)],
        out_hbm.at[cid, pl.ds(sid * STRIPE, STRIPE)],
    )


def _k1(x, W1, parts):
    def body(x_ref, w_ref, p_ref, f_ref, dis_ref):
        deg = p_ref[0][:, 0:1] + p_ref[1][:, 0:1] - 1.0
        dis = lax.rsqrt(deg)
        h = jnp.dot(x_ref[...], w_ref[...], preferred_element_type=jnp.float32)
        f_ref[...] = dis * h
        dis_ref[...] = dis

    return pl.pallas_call(
        body,
        grid=(GRID,),
        in_specs=[
            pl.BlockSpec((BN, DIM_IN), lambda i: (i, 0)),
            pl.BlockSpec((DIM_IN, H1), lambda i: (0, 0)),
            pl.BlockSpec((2, BN, DEGW), lambda i: (0, i, 0)),
        ],
        out_specs=[
            pl.BlockSpec((BN, H1), lambda i: (i, 0)),
            pl.BlockSpec((BN, 1), lambda i: (i, 0)),
        ],
        out_shape=[
            jax.ShapeDtypeStruct((N2, H1), jnp.float32),
            jax.ShapeDtypeStruct((N2, 1), jnp.float32),
        ],
    )(x, W1, parts)


def _k2(t1, f1, dis, b1):
    def body(t_ref, f_ref, d_ref, b_ref, o_ref):
        s = t_ref[0] + t_ref[1] - f_ref[...]
        h = jnp.maximum(d_ref[...] * s + b_ref[...], 0.0)
        o_ref[...] = d_ref[...] * h

    return pl.pallas_call(
        body,
        grid=(GRID,),
        in_specs=[
            pl.BlockSpec((2, BN, H1), lambda i: (0, i, 0)),
            pl.BlockSpec((BN, H1), lambda i: (i, 0)),
            pl.BlockSpec((BN, 1), lambda i: (i, 0)),
            pl.BlockSpec((1, H1), lambda i: (0, 0)),
        ],
        out_specs=pl.BlockSpec((BN, H1), lambda i: (i, 0)),
        out_shape=jax.ShapeDtypeStruct((N2, H1), jnp.float32),
    )(t1, f1, dis, b1)


def _k3(t2, g, dis, W2, b2, W_lin, b_lin):
    def body(t_ref, g_ref, d_ref, w2_ref, b_ref, w_ref, bl_ref, lg_ref,
             pr_ref):
        s = d_ref[...] * (t_ref[0] + t_ref[1] - g_ref[...])
        h = jnp.maximum(
            jnp.dot(s, w2_ref[...], preferred_element_type=jnp.float32)
            + b_ref[...], 0.0)
        lg = jnp.dot(h, w_ref[...], preferred_element_type=jnp.float32)
        lg = lg + bl_ref[...]
        lg_ref[...] = lg
        m = jnp.max(lg, axis=1, keepdims=True)
        e = jnp.exp(lg - m)
        pr_ref[...] = e / jnp.sum(e, axis=1, keepdims=True)

    return pl.pallas_call(
        body,
        grid=(GRID,),
        in_specs=[
            pl.BlockSpec((2, BN, H1), lambda i: (0, i, 0)),
            pl.BlockSpec((BN, H1), lambda i: (i, 0)),
            pl.BlockSpec((BN, 1), lambda i: (i, 0)),
            pl.BlockSpec((H1, H2), lambda i: (0, 0)),
            pl.BlockSpec((1, H2), lambda i: (0, 0)),
            pl.BlockSpec((H2, DIM_OUT), lambda i: (0, 0)),
            pl.BlockSpec((1, DIM_OUT), lambda i: (0, 0)),
        ],
        out_specs=[
            pl.BlockSpec((BN, DIM_OUT), lambda i: (i, 0)),
            pl.BlockSpec((BN, DIM_OUT), lambda i: (i, 0)),
        ],
        out_shape=[
            jax.ShapeDtypeStruct((N_NODES, DIM_OUT), jnp.float32),
            jax.ShapeDtypeStruct((N_NODES, DIM_OUT), jnp.float32),
        ],
    )(t2, g, dis, W2, b2, W_lin, b_lin)


_agg128 = _make_agg(H1)


def kernel(x, edge_index, W1, b1, W2, b2, W_lin, b_lin):
    src = edge_index[0].astype(jnp.int32)
    dst = edge_index[1].astype(jnp.int32)
    pad = EPAD - N_EDGES
    src_p = jnp.concatenate([src, jnp.zeros((pad,), jnp.int32)]).reshape(
        NS * (NCH0 + NCH1), C)
    pad_dst = N_NODES + jnp.arange(pad, dtype=jnp.int32) % (N2 - N_NODES)
    dst_p = jnp.concatenate([dst, pad_dst]).reshape(NS * (NCH0 + NCH1), C)
    counts = [NCH0 if w % NC == 0 else NCH1 for w in range(NW)]
    offs = [0]
    for n in counts:
        offs.append(offs[-1] + n)
    src_t = jnp.stack([
        jnp.pad(src_p[offs[w]:offs[w] + counts[w]],
                ((0, NCHM - counts[w]), (0, 0))) for w in range(NW)])
    dst_t = jnp.stack([
        jnp.pad(dst_p[offs[w]:offs[w] + counts[w]],
                ((0, NCHM - counts[w]), (0, 0))) for w in range(NW)])
    ones16 = jnp.ones((N2, DEGW), jnp.float32)

    parts = _deg(dst_t, ones16)
    f1, dis = _k1(x, W1, parts)
    t1 = _agg128(f1, src_t, dst_t)
    g = _k2(t1, f1, dis, b1.reshape(1, H1))
    t2 = _agg128(g, src_t, dst_t)
    logits, probs = _k3(t2, g, dis, W2, b2.reshape(1, H2), W_lin,
                        b_lin.reshape(1, DIM_OUT))
    return (logits, probs)


# speedup vs baseline: 1.0836x; 1.0836x over previous
import functools

import jax
import jax.numpy as jnp
from jax import lax
from jax.experimental import pallas as pl
from jax.experimental.pallas import tpu as pltpu
from jax.experimental.pallas import tpu_sc as plsc

N_NODES = 10000
N_EDGES = 320000
DIM_IN = 128
H1 = 128
H2 = 64
DIM_OUT = 16

NC = 2
NS = 16
NW = NC * NS
C = 128
NCH0 = 101
NCH1 = 59
NCHM = max(NCH0, NCH1)
EPAD = NS * (NCH0 + NCH1) * C
N2 = 10112
STRIPE = N2 // NS

BN = 2528
GRID = 4

_MESH = dict(core_axis_name="c", subcore_axis_name="s")
DEGW = 128
NSTR = 4


def _make_agg(D):
    @functools.partial(
        pl.kernel,
        mesh=plsc.VectorSubcoreMesh(**_MESH),
        out_type=jax.ShapeDtypeStruct((NC, N2, D), jnp.float32),
        scratch_types=[
            pltpu.VMEM((NCHM, C), jnp.int32),
            pltpu.VMEM((NCHM, C), jnp.int32),
            pltpu.VMEM((C, D), jnp.float32),
            pltpu.VMEM_SHARED((N2, D), jnp.float32),
        ] + [pltpu.SemaphoreType.DMA] * NSTR,
    )
    def agg(f_hbm, src_hbm, dst_hbm, out_hbm, src_v, dst_v, buf0,
            table, *gsems):
        cid = lax.axis_index("c")
        sid = lax.axis_index("s")
        wid = sid * NC + cid
        pltpu.sync_copy(
            f_hbm.at[pl.ds(sid * STRIPE, STRIPE)],
            table.at[pl.ds(sid * STRIPE, STRIPE)],
        )
        plsc.subcore_barrier()

        pltpu.sync_copy(src_hbm.at[wid], src_v)
        pltpu.sync_copy(dst_hbm.at[wid], dst_v)

        CS = C // NSTR

        def body(k, carry):
            cps = [
                pltpu.async_copy(
                    f_hbm.at[src_v.at[k, pl.ds(j * CS, CS)]],
                    buf0.at[pl.ds(j * CS, CS)], gsems[j])
                for j in range(NSTR)
            ]
            for cp in cps:
                cp.wait()
            pltpu.sync_copy(buf0, table.at[dst_v.at[k]], add=True)
            return carry

        nch = jnp.where(cid == 0, NCH0, NCH1)
        lax.fori_loop(0, nch, body, 0)
        plsc.subcore_barrier()
        pltpu.sync_copy(
            table.at[pl.ds(sid * STRIPE, STRIPE)],
            out_hbm.at[cid, pl.ds(sid * STRIPE, STRIPE)],
        )

    return agg


@functools.partial(
    pl.kernel,
    mesh=plsc.VectorSubcoreMesh(**_MESH),
    out_type=jax.ShapeDtypeStruct((NC, N2, DEGW), jnp.float32),
    scratch_types=[
        pltpu.VMEM((NCHM, C), jnp.int32),
        pltpu.VMEM((C, DEGW), jnp.float32),
        pltpu.VMEM_SHARED((N2, DEGW), jnp.float32),
    ],
)
def _deg(dst_hbm, ones_hbm, out_hbm, dst_v, ones_v, table):
    cid = lax.axis_index("c")
    sid = lax.axis_index("s")
    wid = sid * NC + cid
    pltpu.sync_copy(dst_hbm.at[wid], dst_v)
    pltpu.sync_copy(ones_hbm.at[pl.ds(0, C)], ones_v)
    pltpu.sync_copy(
        ones_hbm.at[pl.ds(sid * STRIPE, STRIPE)],
        table.at[pl.ds(sid * STRIPE, STRIPE)],
    )
    plsc.subcore_barrier()

    def body(k, carry):
        pltpu.sync_copy(ones_v, table.at[dst_v.at[k]], add=True)
        return carry

    lax.fori_loop(0, jnp.where(cid == 0, NCH0, NCH1), body, 0)
    plsc.subcore_barrier()
    pltpu.sync_copy(
        table.at[pl.ds(sid * STRIPE, STRIPE)],
        out_hbm.at[cid, pl.ds(sid * STRIPE, STRIPE)],
    )


def _k1(x, W1, parts):
    def body(x_ref, w_ref, p_ref, f_ref, dis_ref):
        deg = p_ref[0][:, 0:1] + p_ref[1][:, 0:1] - 1.0
        dis = lax.rsqrt(deg)
        h = jnp.dot(x_ref[...], w_ref[...], preferred_element_type=jnp.float32)
        f_ref[...] = dis * h
        dis_ref[...] = dis

    return pl.pallas_call(
        body,
        grid=(GRID,),
        in_specs=[
            pl.BlockSpec((BN, DIM_IN), lambda i: (i, 0)),
            pl.BlockSpec((DIM_IN, H1), lambda i: (0, 0)),
            pl.BlockSpec((2, BN, DEGW), lambda i: (0, i, 0)),
        ],
        out_specs=[
            pl.BlockSpec((BN, H1), lambda i: (i, 0)),
            pl.BlockSpec((BN, 1), lambda i: (i, 0)),
        ],
        out_shape=[
            jax.ShapeDtypeStruct((N2, H1), jnp.float32),
            jax.ShapeDtypeStruct((N2, 1), jnp.float32),
        ],
    )(x, W1, parts)


def _k2(t1, f1, dis, b1):
    def body(t_ref, f_ref, d_ref, b_ref, o_ref):
        s = t_ref[0] + t_ref[1] - f_ref[...]
        h = jnp.maximum(d_ref[...] * s + b_ref[...], 0.0)
        o_ref[...] = d_ref[...] * h

    return pl.pallas_call(
        body,
        grid=(GRID,),
        in_specs=[
            pl.BlockSpec((2, BN, H1), lambda i: (0, i, 0)),
            pl.BlockSpec((BN, H1), lambda i: (i, 0)),
            pl.BlockSpec((BN, 1), lambda i: (i, 0)),
            pl.BlockSpec((1, H1), lambda i: (0, 0)),
        ],
        out_specs=pl.BlockSpec((BN, H1), lambda i: (i, 0)),
        out_shape=jax.ShapeDtypeStruct((N2, H1), jnp.float32),
    )(t1, f1, dis, b1)


def _k3(t2, g, dis, W2, b2, W_lin, b_lin):
    def body(t_ref, g_ref, d_ref, w2_ref, b_ref, w_ref, bl_ref, lg_ref,
             pr_ref):
        s = d_ref[...] * (t_ref[0] + t_ref[1] - g_ref[...])
        h = jnp.maximum(
            jnp.dot(s, w2_ref[...], preferred_element_type=jnp.float32)
            + b_ref[...], 0.0)
        lg = jnp.dot(h, w_ref[...], preferred_element_type=jnp.float32)
        lg = lg + bl_ref[...]
        lg_ref[...] = lg
        m = jnp.max(lg, axis=1, keepdims=True)
        e = jnp.exp(lg - m)
        pr_ref[...] = e / jnp.sum(e, axis=1, keepdims=True)

    return pl.pallas_call(
        body,
        grid=(GRID,),
        in_specs=[
            pl.BlockSpec((2, BN, H1), lambda i: (0, i, 0)),
            pl.BlockSpec((BN, H1), lambda i: (i, 0)),
            pl.BlockSpec((BN, 1), lambda i: (i, 0)),
            pl.BlockSpec((H1, H2), lambda i: (0, 0)),
            pl.BlockSpec((1, H2), lambda i: (0, 0)),
            pl.BlockSpec((H2, DIM_OUT), lambda i: (0, 0)),
            pl.BlockSpec((1, DIM_OUT), lambda i: (0, 0)),
        ],
        out_specs=[
            pl.BlockSpec((BN, DIM_OUT), lambda i: (i, 0)),
            pl.BlockSpec((BN, DIM_OUT), lambda i: (i, 0)),
        ],
        out_shape=[
            jax.ShapeDtypeStruct((N_NODES, DIM_OUT), jnp.float32),
            jax.ShapeDtypeStruct((N_NODES, DIM_OUT), jnp.float32),
        ],
    )(t2, g, dis, W2, b2, W_lin, b_lin)


_agg128 = _make_agg(H1)


def kernel(x, edge_index, W1, b1, W2, b2, W_lin, b_lin):
    src = edge_index[0].astype(jnp.int32)
    dst = edge_index[1].astype(jnp.int32)
    pad = EPAD - N_EDGES
    src_p = jnp.concatenate([src, jnp.zeros((pad,), jnp.int32)]).reshape(
        NS * (NCH0 + NCH1), C)
    pad_dst = N_NODES + jnp.arange(pad, dtype=jnp.int32) % (N2 - N_NODES)
    dst_p = jnp.concatenate([dst, pad_dst]).reshape(NS * (NCH0 + NCH1), C)
    counts = [NCH0 if w % NC == 0 else NCH1 for w in range(NW)]
    offs = [0]
    for n in counts:
        offs.append(offs[-1] + n)
    src_t = jnp.stack([
        jnp.pad(src_p[offs[w]:offs[w] + counts[w]],
                ((0, NCHM - counts[w]), (0, 0))) for w in range(NW)])
    dst_t = jnp.stack([
        jnp.pad(dst_p[offs[w]:offs[w] + counts[w]],
                ((0, NCHM - counts[w]), (0, 0))) for w in range(NW)])
    ones16 = jnp.ones((N2, DEGW), jnp.float32)

    parts = _deg(dst_t, ones16)
    f1, dis = _k1(x, W1, parts)
    t1 = _agg128(f1, src_t, dst_t)
    g = _k2(t1, f1, dis, b1.reshape(1, H1))
    t2 = _agg128(g, src_t, dst_t)
    logits, probs = _k3(t2, g, dis, W2, b2.reshape(1, H2), W_lin,
                        b_lin.reshape(1, DIM_OUT))
    return (logits, probs)


# speedup vs baseline: 1.4174x; 1.3080x over previous
import functools

import jax
import jax.numpy as jnp
from jax import lax
from jax.experimental import pallas as pl
from jax.experimental.pallas import tpu as pltpu
from jax.experimental.pallas import tpu_sc as plsc

N_NODES = 10000
N_EDGES = 320000
DIM_IN = 128
H1 = 128
H2 = 64
DIM_OUT = 16

NC = 2
NS = 16
NW = NC * NS
C = 128
NCH0 = 79
NCH1 = 79
NCHM = max(NCH0, NCH1)
EPAD = NS * (NCH0 + NCH1) * C
N2 = 10112
STRIPE = N2 // NS

BN = 2528
GRID = 4

_MESH = dict(core_axis_name="c", subcore_axis_name="s")
DEGW = 128
NSTR = 4


def _make_agg(D):
    @functools.partial(
        pl.kernel,
        mesh=plsc.VectorSubcoreMesh(**_MESH),
        out_type=jax.ShapeDtypeStruct((NC, N2, D), jnp.float32),
        scratch_types=[
            pltpu.VMEM((NCHM, C), jnp.int32),
            pltpu.VMEM((NCHM, C), jnp.int32),
            pltpu.VMEM((C, D), jnp.float32),
            pltpu.VMEM_SHARED((N2, D), jnp.float32),
        ] + [pltpu.SemaphoreType.DMA] * NSTR,
    )
    def agg(f_hbm, src_hbm, dst_hbm, out_hbm, src_v, dst_v, buf0,
            table, *gsems):
        cid = lax.axis_index("c")
        sid = lax.axis_index("s")
        wid = sid * NC + cid
        pltpu.sync_copy(
            f_hbm.at[pl.ds(sid * STRIPE, STRIPE)],
            table.at[pl.ds(sid * STRIPE, STRIPE)],
        )
        plsc.subcore_barrier()

        pltpu.sync_copy(src_hbm.at[wid], src_v)
        pltpu.sync_copy(dst_hbm.at[wid], dst_v)

        CS = C // NSTR

        def body(k, carry):
            cps = [
                pltpu.async_copy(
                    f_hbm.at[src_v.at[k, pl.ds(j * CS, CS)]],
                    buf0.at[pl.ds(j * CS, CS)], gsems[j])
                for j in range(NSTR)
            ]
            for cp in cps:
                cp.wait()
            pltpu.sync_copy(buf0, table.at[dst_v.at[k]], add=True)
            return carry

        nch = NCH0 if NCH0 == NCH1 else jnp.where(cid == 0, NCH0, NCH1)
        lax.fori_loop(0, nch, body, 0)
        plsc.subcore_barrier()
        pltpu.sync_copy(
            table.at[pl.ds(sid * STRIPE, STRIPE)],
            out_hbm.at[cid, pl.ds(sid * STRIPE, STRIPE)],
        )

    return agg


@functools.partial(
    pl.kernel,
    mesh=plsc.VectorSubcoreMesh(**_MESH),
    out_type=jax.ShapeDtypeStruct((NC, N2, DEGW), jnp.float32),
    scratch_types=[
        pltpu.VMEM((NCHM, C), jnp.int32),
        pltpu.VMEM((C, DEGW), jnp.float32),
        pltpu.VMEM_SHARED((N2, DEGW), jnp.float32),
    ],
)
def _deg(dst_hbm, ones_hbm, out_hbm, dst_v, ones_v, table):
    cid = lax.axis_index("c")
    sid = lax.axis_index("s")
    wid = sid * NC + cid
    pltpu.sync_copy(dst_hbm.at[wid], dst_v)
    pltpu.sync_copy(ones_hbm.at[pl.ds(0, C)], ones_v)
    pltpu.sync_copy(
        ones_hbm.at[pl.ds(sid * STRIPE, STRIPE)],
        table.at[pl.ds(sid * STRIPE, STRIPE)],
    )
    plsc.subcore_barrier()

    def body(k, carry):
        pltpu.sync_copy(ones_v, table.at[dst_v.at[k]], add=True)
        return carry

    lax.fori_loop(0, NCH0 if NCH0 == NCH1 else jnp.where(cid == 0, NCH0, NCH1), body, 0)
    plsc.subcore_barrier()
    pltpu.sync_copy(
        table.at[pl.ds(sid * STRIPE, STRIPE)],
        out_hbm.at[cid, pl.ds(sid * STRIPE, STRIPE)],
    )


def _k1(x, W1, parts):
    def body(x_ref, w_ref, p_ref, f_ref, dis_ref):
        deg = p_ref[0][:, 0:1] + p_ref[1][:, 0:1] - 1.0
        dis = lax.rsqrt(deg)
        h = jnp.dot(x_ref[...], w_ref[...], preferred_element_type=jnp.float32)
        f_ref[...] = dis * h
        dis_ref[...] = dis

    return pl.pallas_call(
        body,
        grid=(GRID,),
        in_specs=[
            pl.BlockSpec((BN, DIM_IN), lambda i: (i, 0)),
            pl.BlockSpec((DIM_IN, H1), lambda i: (0, 0)),
            pl.BlockSpec((2, BN, DEGW), lambda i: (0, i, 0)),
        ],
        out_specs=[
            pl.BlockSpec((BN, H1), lambda i: (i, 0)),
            pl.BlockSpec((BN, 1), lambda i: (i, 0)),
        ],
        out_shape=[
            jax.ShapeDtypeStruct((N2, H1), jnp.float32),
            jax.ShapeDtypeStruct((N2, 1), jnp.float32),
        ],
    )(x, W1, parts)


def _k2(t1, f1, dis, b1):
    def body(t_ref, f_ref, d_ref, b_ref, o_ref):
        s = t_ref[0] + t_ref[1] - f_ref[...]
        h = jnp.maximum(d_ref[...] * s + b_ref[...], 0.0)
        o_ref[...] = d_ref[...] * h

    return pl.pallas_call(
        body,
        grid=(GRID,),
        in_specs=[
            pl.BlockSpec((2, BN, H1), lambda i: (0, i, 0)),
            pl.BlockSpec((BN, H1), lambda i: (i, 0)),
            pl.BlockSpec((BN, 1), lambda i: (i, 0)),
            pl.BlockSpec((1, H1), lambda i: (0, 0)),
        ],
        out_specs=pl.BlockSpec((BN, H1), lambda i: (i, 0)),
        out_shape=jax.ShapeDtypeStruct((N2, H1), jnp.float32),
    )(t1, f1, dis, b1)


def _k3(t2, g, dis, W2, b2, W_lin, b_lin):
    def body(t_ref, g_ref, d_ref, w2_ref, b_ref, w_ref, bl_ref, lg_ref,
             pr_ref):
        s = d_ref[...] * (t_ref[0] + t_ref[1] - g_ref[...])
        h = jnp.maximum(
            jnp.dot(s, w2_ref[...], preferred_element_type=jnp.float32)
            + b_ref[...], 0.0)
        lg = jnp.dot(h, w_ref[...], preferred_element_type=jnp.float32)
        lg = lg + bl_ref[...]
        lg_ref[...] = lg
        m = jnp.max(lg, axis=1, keepdims=True)
        e = jnp.exp(lg - m)
        pr_ref[...] = e / jnp.sum(e, axis=1, keepdims=True)

    return pl.pallas_call(
        body,
        grid=(GRID,),
        in_specs=[
            pl.BlockSpec((2, BN, H1), lambda i: (0, i, 0)),
            pl.BlockSpec((BN, H1), lambda i: (i, 0)),
            pl.BlockSpec((BN, 1), lambda i: (i, 0)),
            pl.BlockSpec((H1, H2), lambda i: (0, 0)),
            pl.BlockSpec((1, H2), lambda i: (0, 0)),
            pl.BlockSpec((H2, DIM_OUT), lambda i: (0, 0)),
            pl.BlockSpec((1, DIM_OUT), lambda i: (0, 0)),
        ],
        out_specs=[
            pl.BlockSpec((BN, DIM_OUT), lambda i: (i, 0)),
            pl.BlockSpec((BN, DIM_OUT), lambda i: (i, 0)),
        ],
        out_shape=[
            jax.ShapeDtypeStruct((N_NODES, DIM_OUT), jnp.float32),
            jax.ShapeDtypeStruct((N_NODES, DIM_OUT), jnp.float32),
        ],
    )(t2, g, dis, W2, b2, W_lin, b_lin)


_agg128 = _make_agg(H1)


def kernel(x, edge_index, W1, b1, W2, b2, W_lin, b_lin):
    src = edge_index[0].astype(jnp.int32)
    dst = edge_index[1].astype(jnp.int32)
    pad = EPAD - N_EDGES
    src_p = jnp.concatenate([src, jnp.zeros((pad,), jnp.int32)]).reshape(
        NS * (NCH0 + NCH1), C)
    pad_dst = N_NODES + jnp.arange(pad, dtype=jnp.int32) % (N2 - N_NODES)
    dst_p = jnp.concatenate([dst, pad_dst]).reshape(NS * (NCH0 + NCH1), C)
    counts = [NCH0 if w % NC == 0 else NCH1 for w in range(NW)]
    offs = [0]
    for n in counts:
        offs.append(offs[-1] + n)
    src_t = jnp.stack([
        jnp.pad(src_p[offs[w]:offs[w] + counts[w]],
                ((0, NCHM - counts[w]), (0, 0))) for w in range(NW)])
    dst_t = jnp.stack([
        jnp.pad(dst_p[offs[w]:offs[w] + counts[w]],
                ((0, NCHM - counts[w]), (0, 0))) for w in range(NW)])
    ones16 = jnp.ones((N2, DEGW), jnp.float32)

    parts = _deg(dst_t, ones16)
    f1, dis = _k1(x, W1, parts)
    t1 = _agg128(f1, src_t, dst_t)
    g = _k2(t1, f1, dis, b1.reshape(1, H1))
    t2 = _agg128(g, src_t, dst_t)
    logits, probs = _k3(t2, g, dis, W2, b2.reshape(1, H2), W_lin,
                        b_lin.reshape(1, DIM_OUT))
    return (logits, probs)


# speedup vs baseline: 1.5344x; 1.0826x over previous
import functools

import jax
import jax.numpy as jnp
from jax import lax
from jax.experimental import pallas as pl
from jax.experimental.pallas import tpu as pltpu
from jax.experimental.pallas import tpu_sc as plsc

N_NODES = 10000
N_EDGES = 320000
DIM_IN = 128
H1 = 128
H2 = 64
DIM_OUT = 16

NC = 2
NS = 16
NW = NC * NS
C = 128
NCH0 = 79
NCH1 = 79
NCHM = max(NCH0, NCH1)
EPAD = NS * (NCH0 + NCH1) * C
N2 = 10112
STRIPE = N2 // NS

BN = 2528
GRID = 4

_MESH = dict(core_axis_name="c", subcore_axis_name="s")
DEGW = 128
NSTR = 4


def _make_agg(D):
    @functools.partial(
        pl.kernel,
        mesh=plsc.VectorSubcoreMesh(**_MESH),
        out_type=jax.ShapeDtypeStruct((NC, N2, D), jnp.float32),
        scratch_types=[
            pltpu.VMEM((NCHM, C), jnp.int32),
            pltpu.VMEM((NCHM, C), jnp.int32),
            pltpu.VMEM((C, D), jnp.float32),
            pltpu.VMEM_SHARED((N2, D), jnp.float32),
        ] + [pltpu.SemaphoreType.DMA] * NSTR,
    )
    def agg(f_hbm, src_hbm, dst_hbm, out_hbm, src_v, dst_v, buf0,
            table, *gsems):
        cid = lax.axis_index("c")
        sid = lax.axis_index("s")
        wid = sid * NC + cid
        pltpu.sync_copy(
            f_hbm.at[pl.ds(sid * STRIPE, STRIPE)],
            table.at[pl.ds(sid * STRIPE, STRIPE)],
        )
        plsc.subcore_barrier()

        pltpu.sync_copy(src_hbm.at[wid], src_v)
        pltpu.sync_copy(dst_hbm.at[wid], dst_v)

        CS = C // NSTR

        def body(k, carry):
            cps = [
                pltpu.async_copy(
                    f_hbm.at[src_v.at[k, pl.ds(j * CS, CS)]],
                    buf0.at[pl.ds(j * CS, CS)], gsems[j])
                for j in range(NSTR)
            ]
            for cp in cps:
                cp.wait()
            pltpu.sync_copy(buf0, table.at[dst_v.at[k]], add=True)
            return carry

        nch = NCH0 if NCH0 == NCH1 else jnp.where(cid == 0, NCH0, NCH1)
        lax.fori_loop(0, nch, body, 0)
        plsc.subcore_barrier()
        pltpu.sync_copy(
            table.at[pl.ds(sid * STRIPE, STRIPE)],
            out_hbm.at[cid, pl.ds(sid * STRIPE, STRIPE)],
        )

    return agg


@functools.partial(
    pl.kernel,
    mesh=plsc.VectorSubcoreMesh(**_MESH),
    out_type=jax.ShapeDtypeStruct((NC, N2, DEGW), jnp.float32),
    scratch_types=[
        pltpu.VMEM((NCHM, C), jnp.int32),
        pltpu.VMEM((C, DEGW), jnp.float32),
        pltpu.VMEM_SHARED((N2, DEGW), jnp.float32),
    ],
)
def _deg(dst_hbm, ones_hbm, out_hbm, dst_v, ones_v, table):
    cid = lax.axis_index("c")
    sid = lax.axis_index("s")
    wid = sid * NC + cid
    pltpu.sync_copy(dst_hbm.at[wid], dst_v)
    pltpu.sync_copy(ones_hbm.at[pl.ds(0, C)], ones_v)
    pltpu.sync_copy(
        ones_hbm.at[pl.ds(sid * STRIPE, STRIPE)],
        table.at[pl.ds(sid * STRIPE, STRIPE)],
    )
    plsc.subcore_barrier()

    def body(k, carry):
        pltpu.sync_copy(ones_v, table.at[dst_v.at[k]], add=True)
        return carry

    lax.fori_loop(0, NCH0 if NCH0 == NCH1 else jnp.where(cid == 0, NCH0, NCH1), body, 0)
    plsc.subcore_barrier()
    pltpu.sync_copy(
        table.at[pl.ds(sid * STRIPE, STRIPE)],
        out_hbm.at[cid, pl.ds(sid * STRIPE, STRIPE)],
    )


def _k1(x, W1, parts):
    def body(x_ref, w_ref, p_ref, f_ref, dis_ref):
        deg = p_ref[0][:, 0:1] + p_ref[1][:, 0:1] - 1.0
        dis = lax.rsqrt(deg)
        h = jnp.dot(x_ref[...], w_ref[...], preferred_element_type=jnp.float32)
        f_ref[...] = dis * h
        dis_ref[...] = dis

    return pl.pallas_call(
        body,
        grid=(GRID,),
        in_specs=[
            pl.BlockSpec((BN, DIM_IN), lambda i: (i, 0)),
            pl.BlockSpec((DIM_IN, H1), lambda i: (0, 0)),
            pl.BlockSpec((2, BN, DEGW), lambda i: (0, i, 0)),
        ],
        out_specs=[
            pl.BlockSpec((BN, H1), lambda i: (i, 0)),
            pl.BlockSpec((BN, 1), lambda i: (i, 0)),
        ],
        out_shape=[
            jax.ShapeDtypeStruct((N2, H1), jnp.float32),
            jax.ShapeDtypeStruct((N2, 1), jnp.float32),
        ],
    )(x, W1, parts)


def _k2(t1, f1, dis, b1):
    def body(t_ref, f_ref, d_ref, b_ref, o_ref):
        s = t_ref[0] + t_ref[1] - f_ref[...]
        h = jnp.maximum(d_ref[...] * s + b_ref[...], 0.0)
        o_ref[...] = d_ref[...] * h

    return pl.pallas_call(
        body,
        grid=(GRID,),
        in_specs=[
            pl.BlockSpec((2, BN, H1), lambda i: (0, i, 0)),
            pl.BlockSpec((BN, H1), lambda i: (i, 0)),
            pl.BlockSpec((BN, 1), lambda i: (i, 0)),
            pl.BlockSpec((1, H1), lambda i: (0, 0)),
        ],
        out_specs=pl.BlockSpec((BN, H1), lambda i: (i, 0)),
        out_shape=jax.ShapeDtypeStruct((N2, H1), jnp.float32),
    )(t1, f1, dis, b1)


def _k3(t2, g, dis, W2, b2, W_lin, b_lin):
    def body(t_ref, g_ref, d_ref, w2_ref, b_ref, w_ref, bl_ref, lg_ref,
             pr_ref):
        s = d_ref[...] * (t_ref[0] + t_ref[1] - g_ref[...])
        h = jnp.maximum(
            jnp.dot(s, w2_ref[...], preferred_element_type=jnp.float32)
            + b_ref[...], 0.0)
        lg = jnp.dot(h, w_ref[...], preferred_element_type=jnp.float32)
        lg = lg + bl_ref[...]
        lg_ref[...] = lg
        m = jnp.max(lg, axis=1, keepdims=True)
        e = jnp.exp(lg - m)
        pr_ref[...] = e / jnp.sum(e, axis=1, keepdims=True)

    return pl.pallas_call(
        body,
        grid=(GRID,),
        in_specs=[
            pl.BlockSpec((2, BN, H1), lambda i: (0, i, 0)),
            pl.BlockSpec((BN, H1), lambda i: (i, 0)),
            pl.BlockSpec((BN, 1), lambda i: (i, 0)),
            pl.BlockSpec((H1, H2), lambda i: (0, 0)),
            pl.BlockSpec((1, H2), lambda i: (0, 0)),
            pl.BlockSpec((H2, DIM_OUT), lambda i: (0, 0)),
            pl.BlockSpec((1, DIM_OUT), lambda i: (0, 0)),
        ],
        out_specs=[
            pl.BlockSpec((BN, DIM_OUT), lambda i: (i, 0)),
            pl.BlockSpec((BN, DIM_OUT), lambda i: (i, 0)),
        ],
        out_shape=[
            jax.ShapeDtypeStruct((N_NODES, DIM_OUT), jnp.float32),
            jax.ShapeDtypeStruct((N_NODES, DIM_OUT), jnp.float32),
        ],
    )(t2, g, dis, W2, b2, W_lin, b_lin)


_agg128 = _make_agg(H1)


def kernel(x, edge_index, W1, b1, W2, b2, W_lin, b_lin):
    src = edge_index[0].astype(jnp.int32)
    dst = edge_index[1].astype(jnp.int32)
    pad = EPAD - N_EDGES
    src_t = jnp.concatenate([src, jnp.zeros((pad,), jnp.int32)]).reshape(
        NW, NCHM, C)
    pad_dst = N_NODES + jnp.arange(pad, dtype=jnp.int32) % (N2 - N_NODES)
    dst_t = jnp.concatenate([dst, pad_dst]).reshape(NW, NCHM, C)
    ones16 = jnp.ones((N2, DEGW), jnp.float32)

    parts = _deg(dst_t, ones16)
    f1, dis = _k1(x, W1, parts)
    t1 = _agg128(f1, src_t, dst_t)
    g = _k2(t1, f1, dis, b1.reshape(1, H1))
    t2 = _agg128(g, src_t, dst_t)
    logits, probs = _k3(t2, g, dis, W2, b2.reshape(1, H2), W_lin,
                        b_lin.reshape(1, DIM_OUT))
    return (logits, probs)
